# baseline (speedup 1.0000x reference)
import functools

import jax
import jax.numpy as jnp
from jax import lax
from jax.experimental import pallas as pl
from jax.experimental.pallas import tpu as pltpu
from jax.experimental.pallas import tpu_sc as plsc

N = 10000
E = 320000
R = 4
NR = N * R
TRASH = NR
NSEG_PAD = 40448
NSEG128 = NSEG_PAD // 4
NPADT = 10240
NCORES = 2
NSUB = 16
NW = NCORES * NSUB
EW = E // NW
GRB = 256
GROUPS = 40
SBE = GROUPS * GRB
ROWS_PER_TILE = NSEG_PAD // NSUB
ROWS128_PER_TILE = NSEG128 // NSUB
TROWS_PER_TILE = NPADT // NSUB


def _pad_perworker(a, padval):
    a = a.astype(jnp.int32).reshape(NW, EW)
    a = jnp.pad(a, ((0, 0), (0, SBE - EW)), constant_values=padval)
    return a.reshape(NW, GROUPS, GRB)


@functools.partial(jax.jit, static_argnames=("C", "with_counts"))
def _sc_layer(table_cm, gidx, seg, zeros32, ones32, C, with_counts=False):
    mesh = plsc.VectorSubcoreMesh(core_axis_name="c", subcore_axis_name="s")
    outs = [jax.ShapeDtypeStruct((C, NCORES, NSEG_PAD, 32), jnp.float32)]
    if with_counts:
        outs.append(jax.ShapeDtypeStruct((NCORES, NSEG_PAD, 32), jnp.float32))

    @functools.partial(
        pl.kernel,
        out_type=outs,
        mesh=mesh,
        scratch_types=[
            pltpu.VMEM((GROUPS, GRB), jnp.int32),
            pltpu.VMEM((GROUPS, GRB), jnp.int32),
            pltpu.VMEM((GRB, 32), jnp.float32),
            pltpu.VMEM_SHARED((NSEG_PAD, 32), jnp.float32),
            pltpu.VMEM_SHARED((NPADT, 32), jnp.float32),
        ],
        compiler_params=pltpu.CompilerParams(use_tc_tiling_on_sc=False),
    )
    def k(table_hbm, gidx_hbm, seg_hbm, zeros_hbm, ones_hbm, *rest):
        if with_counts:
            out_hbm, cnt_hbm, gbuf, sbuf, dbuf, acc, tbl = rest
        else:
            out_hbm, gbuf, sbuf, dbuf, acc, tbl = rest
        cid = lax.axis_index("c")
        sid = lax.axis_index("s")
        wid = sid * NCORES + cid
        row0 = sid * ROWS_PER_TILE
        trow0 = sid * TROWS_PER_TILE
        pltpu.sync_copy(seg_hbm.at[wid], sbuf)
        pltpu.sync_copy(gidx_hbm.at[wid], gbuf)
        if with_counts:
            pltpu.sync_copy(zeros_hbm, acc.at[pl.ds(row0, ROWS_PER_TILE)])
            pltpu.sync_copy(ones_hbm, dbuf)
            plsc.subcore_barrier()
            for grp in range(GROUPS):
                pltpu.sync_copy(dbuf, acc.at[sbuf.at[grp]], add=True)
            plsc.subcore_barrier()
            pltpu.sync_copy(
                acc.at[pl.ds(row0, ROWS_PER_TILE)],
                cnt_hbm.at[cid].at[pl.ds(row0, ROWS_PER_TILE)],
            )
            plsc.subcore_barrier()
        for c in range(C):
            pltpu.sync_copy(zeros_hbm, acc.at[pl.ds(row0, ROWS_PER_TILE)])
            pltpu.sync_copy(
                table_hbm.at[c].at[pl.ds(trow0, TROWS_PER_TILE)],
                tbl.at[pl.ds(trow0, TROWS_PER_TILE)],
            )
            plsc.subcore_barrier()
            for grp in range(GROUPS):
                pltpu.sync_copy(tbl.at[gbuf.at[grp]], dbuf)
                pltpu.sync_copy(dbuf, acc.at[sbuf.at[grp]], add=True)
            plsc.subcore_barrier()
            pltpu.sync_copy(
                acc.at[pl.ds(row0, ROWS_PER_TILE)],
                out_hbm.at[c].at[cid].at[pl.ds(row0, ROWS_PER_TILE)],
            )
            plsc.subcore_barrier()

    return k(table_cm, gidx, seg, zeros32, ones32)


_NB = 10
_NBLK = N // _NB


def _tc_post(parts, cnt, W, x, root, b, O, chunk_major_out):
    C = parts.shape[0]
    F_in = x.shape[1]
    CO = O // 32

    def body(p_ref, c_ref, w_ref, x_ref, r_ref, b_ref, o_ref, *ocm):
        inv = 1.0 / jnp.maximum(c_ref[0] + c_ref[1], 1.0)
        acc = jnp.dot(x_ref[...], r_ref[...],
                      preferred_element_type=jnp.float32)
        for c in range(C):
            p128 = p_ref[c, 0] + p_ref[c, 1]
            for r in range(R):
                m = (p128[:, r * 32:(r + 1) * 32]
                     * inv[:, r * 32:(r + 1) * 32])
                acc = acc + jnp.dot(
                    m, w_ref[r, c * 32:(c + 1) * 32, :],
                    preferred_element_type=jnp.float32)
        h = jnp.maximum(acc + b_ref[...], 0.0)
        o_ref[...] = h
        if ocm:
            for c in range(CO):
                ocm[0][c] = h[:, c * 32:(c + 1) * 32]

    in_specs = [
        pl.BlockSpec((C, 2, _NBLK, 128), lambda nb: (0, 0, nb, 0)),
        pl.BlockSpec((2, _NBLK, 128), lambda nb: (0, nb, 0)),
        pl.BlockSpec((R, 32 * C, O), lambda nb: (0, 0, 0)),
        pl.BlockSpec((_NBLK, F_in), lambda nb: (nb, 0)),
        pl.BlockSpec((F_in, O), lambda nb: (0, 0)),
        pl.BlockSpec((1, O), lambda nb: (0, 0)),
    ]
    out_specs = [pl.BlockSpec((_NBLK, O), lambda nb: (nb, 0))]
    out_shape = [jax.ShapeDtypeStruct((N, O), jnp.float32)]
    if chunk_major_out:
        out_specs.append(pl.BlockSpec((CO, _NBLK, 32), lambda nb: (0, nb, 0)))
        out_shape.append(jax.ShapeDtypeStruct((CO, NPADT, 32), jnp.float32))

    return pl.pallas_call(
        body,
        grid=(_NB,),
        in_specs=in_specs,
        out_specs=out_specs,
        out_shape=out_shape,
    )(parts, cnt, W, x, root, b.reshape(1, O))


def kernel(atom, bond, connection, W1, root1, b1, W2, root2, b2, W3, root3, b3):
    src = connection[0].astype(jnp.int32)
    dst = connection[1].astype(jnp.int32)
    etype = bond.astype(jnp.int32)

    gidx = _pad_perworker(src, 0)
    seg = _pad_perworker(dst * R + etype, TRASH)

    zeros32 = jnp.zeros((ROWS_PER_TILE, 32), jnp.float32)
    ones32 = jnp.ones((GRB, 32), jnp.float32)

    atom_cm = jnp.pad(
        jnp.transpose(atom.reshape(N, 4, 32), (1, 0, 2)),
        ((0, 0), (0, NPADT - N), (0, 0)))

    parts1, cntp = _sc_layer(atom_cm, gidx, seg, zeros32, ones32, C=4,
                             with_counts=True)
    cnt = cntp.reshape(NCORES, NSEG128, 128)
    h1, h1_cm = _tc_post(parts1.reshape(4, NCORES, NSEG128, 128), cnt,
                         W1, atom, root1, b1, O=64, chunk_major_out=True)
    (parts2,) = _sc_layer(h1_cm, gidx, seg, zeros32, ones32, C=2)
    h2, h2_cm = _tc_post(parts2.reshape(2, NCORES, NSEG128, 128), cnt,
                         W2, h1, root2, b2, O=128, chunk_major_out=True)
    (parts3,) = _sc_layer(h2_cm, gidx, seg, zeros32, ones32, C=4)
    (h3,) = _tc_post(parts3.reshape(4, NCORES, NSEG128, 128), cnt,
                     W3, h2, root3, b3, O=256, chunk_major_out=False)
    return h3


# speedup vs baseline: 1.0310x; 1.0310x over previous
import functools

import jax
import jax.numpy as jnp
from jax import lax
from jax.experimental import pallas as pl
from jax.experimental.pallas import tpu as pltpu
from jax.experimental.pallas import tpu_sc as plsc

N = 10000
E = 320000
R = 4
NR = N * R
TRASH = NR
NSEG_PAD = 40448
NSEG128 = NSEG_PAD // 4
NPADT = 10240
NCORES = 2
NSUB = 16
NW = NCORES * NSUB
EW = E // NW
GRB = 256
GROUPS = 40
SBE = GROUPS * GRB
ROWS_PER_TILE = NSEG_PAD // NSUB
ROWS128_PER_TILE = NSEG128 // NSUB
TROWS_PER_TILE = NPADT // NSUB


def _pad_perworker(a, padval):
    a = a.astype(jnp.int32).reshape(NW, EW)
    a = jnp.pad(a, ((0, 0), (0, SBE - EW)), constant_values=padval)
    return a.reshape(NW, GROUPS, GRB)


@functools.partial(jax.jit, static_argnames=("C",))
def _sc_layer(table_cm, gidx, seg, zeros32, C):
    mesh = plsc.VectorSubcoreMesh(core_axis_name="c", subcore_axis_name="s")

    @functools.partial(
        pl.kernel,
        out_type=jax.ShapeDtypeStruct((C, NCORES, NSEG_PAD, 32), jnp.float32),
        mesh=mesh,
        scratch_types=[
            pltpu.VMEM((GROUPS, GRB), jnp.int32),
            pltpu.VMEM((GROUPS, GRB), jnp.int32),
            pltpu.VMEM((GRB, 32), jnp.float32),
            pltpu.VMEM_SHARED((NSEG_PAD, 32), jnp.float32),
            pltpu.VMEM_SHARED((NPADT, 32), jnp.float32),
        ],
        compiler_params=pltpu.CompilerParams(use_tc_tiling_on_sc=False),
    )
    def k(table_hbm, gidx_hbm, seg_hbm, zeros_hbm, out_hbm, gbuf, sbuf, dbuf,
          acc, tbl):
        cid = lax.axis_index("c")
        sid = lax.axis_index("s")
        wid = sid * NCORES + cid
        row0 = sid * ROWS_PER_TILE
        trow0 = sid * TROWS_PER_TILE
        pltpu.sync_copy(seg_hbm.at[wid], sbuf)
        pltpu.sync_copy(gidx_hbm.at[wid], gbuf)
        for c in range(C):
            pltpu.sync_copy(zeros_hbm, acc.at[pl.ds(row0, ROWS_PER_TILE)])
            pltpu.sync_copy(
                table_hbm.at[c].at[pl.ds(trow0, TROWS_PER_TILE)],
                tbl.at[pl.ds(trow0, TROWS_PER_TILE)],
            )
            plsc.subcore_barrier()
            for grp in range(GROUPS):
                pltpu.sync_copy(tbl.at[gbuf.at[grp]], dbuf)
                pltpu.sync_copy(dbuf, acc.at[sbuf.at[grp]], add=True)
            plsc.subcore_barrier()
            pltpu.sync_copy(
                acc.at[pl.ds(row0, ROWS_PER_TILE)],
                out_hbm.at[c].at[cid].at[pl.ds(row0, ROWS_PER_TILE)],
            )
            plsc.subcore_barrier()

    return k(table_cm, gidx, seg, zeros32)


@jax.jit
def _sc_counts(seg, ones32, zeros32):
    mesh = plsc.VectorSubcoreMesh(core_axis_name="c", subcore_axis_name="s")

    @functools.partial(
        pl.kernel,
        out_type=jax.ShapeDtypeStruct((NCORES, NSEG_PAD, 32), jnp.float32),
        mesh=mesh,
        scratch_types=[
            pltpu.VMEM((GROUPS, GRB), jnp.int32),
            pltpu.VMEM((GRB, 32), jnp.float32),
            pltpu.VMEM_SHARED((NSEG_PAD, 32), jnp.float32),
        ],
        compiler_params=pltpu.CompilerParams(use_tc_tiling_on_sc=False),
    )
    def k(seg_hbm, ones_hbm, zeros_hbm, out_hbm, sbuf, obuf, acc):
        cid = lax.axis_index("c")
        sid = lax.axis_index("s")
        wid = sid * NCORES + cid
        row0 = sid * ROWS_PER_TILE
        pltpu.sync_copy(seg_hbm.at[wid], sbuf)
        pltpu.sync_copy(ones_hbm, obuf)
        pltpu.sync_copy(zeros_hbm, acc.at[pl.ds(row0, ROWS_PER_TILE)])
        plsc.subcore_barrier()
        for grp in range(GROUPS):
            pltpu.sync_copy(obuf, acc.at[sbuf.at[grp]], add=True)
        plsc.subcore_barrier()
        pltpu.sync_copy(
            acc.at[pl.ds(row0, ROWS_PER_TILE)],
            out_hbm.at[cid].at[pl.ds(row0, ROWS_PER_TILE)],
        )

    return k(seg, ones32, zeros32)


_NB = 10
_NBLK = N // _NB


def _tc_post(parts, cnt, W, x, root, b, O, chunk_major_out):
    C = parts.shape[0]
    F_in = x.shape[1]
    CO = O // 32

    def body(p_ref, c_ref, w_ref, x_ref, r_ref, b_ref, o_ref, *ocm):
        inv = 1.0 / jnp.maximum(c_ref[0] + c_ref[1], 1.0)
        acc = jnp.dot(x_ref[...], r_ref[...],
                      preferred_element_type=jnp.float32)
        for c in range(C):
            p128 = p_ref[c, 0] + p_ref[c, 1]
            for r in range(R):
                m = (p128[:, r * 32:(r + 1) * 32]
                     * inv[:, r * 32:(r + 1) * 32])
                acc = acc + jnp.dot(
                    m, w_ref[r, c * 32:(c + 1) * 32, :],
                    preferred_element_type=jnp.float32)
        h = jnp.maximum(acc + b_ref[...], 0.0)
        o_ref[...] = h
        if ocm:
            for c in range(CO):
                ocm[0][c] = h[:, c * 32:(c + 1) * 32]

    in_specs = [
        pl.BlockSpec((C, 2, _NBLK, 128), lambda nb: (0, 0, nb, 0)),
        pl.BlockSpec((2, _NBLK, 128), lambda nb: (0, nb, 0)),
        pl.BlockSpec((R, 32 * C, O), lambda nb: (0, 0, 0)),
        pl.BlockSpec((_NBLK, F_in), lambda nb: (nb, 0)),
        pl.BlockSpec((F_in, O), lambda nb: (0, 0)),
        pl.BlockSpec((1, O), lambda nb: (0, 0)),
    ]
    out_specs = [pl.BlockSpec((_NBLK, O), lambda nb: (nb, 0))]
    out_shape = [jax.ShapeDtypeStruct((N, O), jnp.float32)]
    if chunk_major_out:
        out_specs.append(pl.BlockSpec((CO, _NBLK, 32), lambda nb: (0, nb, 0)))
        out_shape.append(jax.ShapeDtypeStruct((CO, NPADT, 32), jnp.float32))

    return pl.pallas_call(
        body,
        grid=(_NB,),
        in_specs=in_specs,
        out_specs=out_specs,
        out_shape=out_shape,
    )(parts, cnt, W, x, root, b.reshape(1, O))


def kernel(atom, bond, connection, W1, root1, b1, W2, root2, b2, W3, root3, b3):
    src = connection[0].astype(jnp.int32)
    dst = connection[1].astype(jnp.int32)
    etype = bond.astype(jnp.int32)

    gidx = _pad_perworker(src, 0)
    seg = _pad_perworker(dst * R + etype, TRASH)

    zeros32 = jnp.zeros((ROWS_PER_TILE, 32), jnp.float32)
    ones32 = jnp.ones((GRB, 32), jnp.float32)

    cnt = _sc_counts(seg, ones32, zeros32).reshape(NCORES, NSEG128, 128)

    atom_cm = jnp.pad(
        jnp.transpose(atom.reshape(N, 4, 32), (1, 0, 2)),
        ((0, 0), (0, NPADT - N), (0, 0)))

    parts1 = _sc_layer(atom_cm, gidx, seg, zeros32, C=4)
    h1, h1_cm = _tc_post(parts1.reshape(4, NCORES, NSEG128, 128), cnt,
                         W1, atom, root1, b1, O=64, chunk_major_out=True)
    parts2 = _sc_layer(h1_cm, gidx, seg, zeros32, C=2)
    h2, h2_cm = _tc_post(parts2.reshape(2, NCORES, NSEG128, 128), cnt,
                         W2, h1, root2, b2, O=128, chunk_major_out=True)
    parts3 = _sc_layer(h2_cm, gidx, seg, zeros32, C=4)
    (h3,) = _tc_post(parts3.reshape(4, NCORES, NSEG128, 128), cnt,
                     W3, h2, root3, b3, O=256, chunk_major_out=False)
    return h3


# speedup vs baseline: 1.0374x; 1.0063x over previous
import functools

import jax
import jax.numpy as jnp
from jax import lax
from jax.experimental import pallas as pl
from jax.experimental.pallas import tpu as pltpu
from jax.experimental.pallas import tpu_sc as plsc

N = 10000
E = 320000
R = 4
NR = N * R
TRASH = NR
NSEG_PAD = 40448
NSEG128 = NSEG_PAD // 4
NPADT = 10240
NCORES = 2
NSUB = 16
NW = NCORES * NSUB
EW = E // NW
GRB = 256
GROUPS = 40
SBE = GROUPS * GRB
ROWS_PER_TILE = NSEG_PAD // NSUB
ROWS128_PER_TILE = NSEG128 // NSUB
TROWS_PER_TILE = NPADT // NSUB


def _pad_perworker(a, padval):
    a = a.astype(jnp.int32).reshape(NW, EW)
    a = jnp.pad(a, ((0, 0), (0, SBE - EW)), constant_values=padval)
    return a.reshape(NW, GROUPS, GRB)


@functools.partial(jax.jit, static_argnames=("C",))
def _sc_layer(table_cm, gidx, seg, zeros32, C):
    mesh = plsc.VectorSubcoreMesh(core_axis_name="c", subcore_axis_name="s")

    @functools.partial(
        pl.kernel,
        out_type=jax.ShapeDtypeStruct((C, NCORES, NSEG_PAD, 32), jnp.float32),
        mesh=mesh,
        scratch_types=[
            pltpu.VMEM((GROUPS, GRB), jnp.int32),
            pltpu.VMEM((GROUPS, GRB), jnp.int32),
            pltpu.VMEM((GRB, 32), jnp.float32),
            pltpu.VMEM_SHARED((NSEG_PAD, 32), jnp.float32),
            pltpu.VMEM_SHARED((NPADT, 32), jnp.float32),
        ],
        compiler_params=pltpu.CompilerParams(use_tc_tiling_on_sc=False),
    )
    def k(table_hbm, gidx_hbm, seg_hbm, zeros_hbm, out_hbm, gbuf, sbuf, dbuf,
          acc, tbl):
        cid = lax.axis_index("c")
        sid = lax.axis_index("s")
        wid = sid * NCORES + cid
        row0 = sid * ROWS_PER_TILE
        trow0 = sid * TROWS_PER_TILE
        pltpu.sync_copy(seg_hbm.at[wid], sbuf)
        pltpu.sync_copy(gidx_hbm.at[wid], gbuf)
        for c in range(C):
            pltpu.sync_copy(zeros_hbm, acc.at[pl.ds(row0, ROWS_PER_TILE)])
            pltpu.sync_copy(
                table_hbm.at[c].at[pl.ds(trow0, TROWS_PER_TILE)],
                tbl.at[pl.ds(trow0, TROWS_PER_TILE)],
            )
            plsc.subcore_barrier()
            for grp in range(GROUPS):
                pltpu.sync_copy(tbl.at[gbuf.at[grp]], dbuf)
                pltpu.sync_copy(dbuf, acc.at[sbuf.at[grp]], add=True)
            plsc.subcore_barrier()
            pltpu.sync_copy(
                acc.at[pl.ds(row0, ROWS_PER_TILE)],
                out_hbm.at[c].at[cid].at[pl.ds(row0, ROWS_PER_TILE)],
            )
            plsc.subcore_barrier()

    return k(table_cm, gidx, seg, zeros32)


@jax.jit
def _sc_counts(seg, ones32, zeros32):
    mesh = plsc.VectorSubcoreMesh(core_axis_name="c", subcore_axis_name="s")

    @functools.partial(
        pl.kernel,
        out_type=jax.ShapeDtypeStruct((NCORES, NSEG_PAD, 32), jnp.float32),
        mesh=mesh,
        scratch_types=[
            pltpu.VMEM((GROUPS, GRB), jnp.int32),
            pltpu.VMEM((GRB, 32), jnp.float32),
            pltpu.VMEM_SHARED((NSEG_PAD, 32), jnp.float32),
        ],
        compiler_params=pltpu.CompilerParams(use_tc_tiling_on_sc=False),
    )
    def k(seg_hbm, ones_hbm, zeros_hbm, out_hbm, sbuf, obuf, acc):
        cid = lax.axis_index("c")
        sid = lax.axis_index("s")
        wid = sid * NCORES + cid
        row0 = sid * ROWS_PER_TILE
        pltpu.sync_copy(seg_hbm.at[wid], sbuf)
        pltpu.sync_copy(ones_hbm, obuf)
        pltpu.sync_copy(zeros_hbm, acc.at[pl.ds(row0, ROWS_PER_TILE)])
        plsc.subcore_barrier()
        for grp in range(GROUPS):
            pltpu.sync_copy(obuf, acc.at[sbuf.at[grp]], add=True)
        plsc.subcore_barrier()
        pltpu.sync_copy(
            acc.at[pl.ds(row0, ROWS_PER_TILE)],
            out_hbm.at[cid].at[pl.ds(row0, ROWS_PER_TILE)],
        )

    return k(seg, ones32, zeros32)


_NB = 5
_NBLK = N // _NB


def _tc_post(parts, cnt, W, x, root, b, O, chunk_major_out):
    C = parts.shape[0]
    F_in = x.shape[1]
    CO = O // 32

    def body(p_ref, c_ref, w_ref, x_ref, r_ref, b_ref, o_ref, *ocm):
        inv = 1.0 / jnp.maximum(c_ref[0] + c_ref[1], 1.0)
        acc = jnp.dot(x_ref[...], r_ref[...],
                      preferred_element_type=jnp.float32)
        for c in range(C):
            p128 = p_ref[c, 0] + p_ref[c, 1]
            for r in range(R):
                m = (p128[:, r * 32:(r + 1) * 32]
                     * inv[:, r * 32:(r + 1) * 32])
                acc = acc + jnp.dot(
                    m, w_ref[r, c * 32:(c + 1) * 32, :],
                    preferred_element_type=jnp.float32)
        h = jnp.maximum(acc + b_ref[...], 0.0)
        o_ref[...] = h
        if ocm:
            for c in range(CO):
                ocm[0][c] = h[:, c * 32:(c + 1) * 32]

    in_specs = [
        pl.BlockSpec((C, 2, _NBLK, 128), lambda nb: (0, 0, nb, 0)),
        pl.BlockSpec((2, _NBLK, 128), lambda nb: (0, nb, 0)),
        pl.BlockSpec((R, 32 * C, O), lambda nb: (0, 0, 0)),
        pl.BlockSpec((_NBLK, F_in), lambda nb: (nb, 0)),
        pl.BlockSpec((F_in, O), lambda nb: (0, 0)),
        pl.BlockSpec((1, O), lambda nb: (0, 0)),
    ]
    out_specs = [pl.BlockSpec((_NBLK, O), lambda nb: (nb, 0))]
    out_shape = [jax.ShapeDtypeStruct((N, O), jnp.float32)]
    if chunk_major_out:
        out_specs.append(pl.BlockSpec((CO, _NBLK, 32), lambda nb: (0, nb, 0)))
        out_shape.append(jax.ShapeDtypeStruct((CO, NPADT, 32), jnp.float32))

    return pl.pallas_call(
        body,
        grid=(_NB,),
        in_specs=in_specs,
        out_specs=out_specs,
        out_shape=out_shape,
    )(parts, cnt, W, x, root, b.reshape(1, O))


def kernel(atom, bond, connection, W1, root1, b1, W2, root2, b2, W3, root3, b3):
    src = connection[0].astype(jnp.int32)
    dst = connection[1].astype(jnp.int32)
    etype = bond.astype(jnp.int32)

    gidx = _pad_perworker(src, 0)
    seg = _pad_perworker(dst * R + etype, TRASH)

    zeros32 = jnp.zeros((ROWS_PER_TILE, 32), jnp.float32)
    ones32 = jnp.ones((GRB, 32), jnp.float32)

    cnt = _sc_counts(seg, ones32, zeros32).reshape(NCORES, NSEG128, 128)

    atom_cm = jnp.pad(
        jnp.transpose(atom.reshape(N, 4, 32), (1, 0, 2)),
        ((0, 0), (0, NPADT - N), (0, 0)))

    parts1 = _sc_layer(atom_cm, gidx, seg, zeros32, C=4)
    h1, h1_cm = _tc_post(parts1.reshape(4, NCORES, NSEG128, 128), cnt,
                         W1, atom, root1, b1, O=64, chunk_major_out=True)
    parts2 = _sc_layer(h1_cm, gidx, seg, zeros32, C=2)
    h2, h2_cm = _tc_post(parts2.reshape(2, NCORES, NSEG128, 128), cnt,
                         W2, h1, root2, b2, O=128, chunk_major_out=True)
    parts3 = _sc_layer(h2_cm, gidx, seg, zeros32, C=4)
    (h3,) = _tc_post(parts3.reshape(4, NCORES, NSEG128, 128), cnt,
                     W3, h2, root3, b3, O=256, chunk_major_out=False)
    return h3
